# Initial kernel scaffold; baseline (speedup 1.0000x reference)
import jax
import jax.numpy as jnp
from jax.experimental import pallas as pl


def kernel(x_in, c1_1_w, c1_1_b, c1_2_w, c1_2_b, c1_3_w, c1_3_b, c2_1_w, c2_1_b, c2_2_w, c2_2_b, c2_3_w, c2_3_b, r1_1_w, r1_1_b, r1_2_w, r1_2_b, r1_3_w, r1_3_b):
    raise NotImplementedError("write your pallas kernel here")


# speedup vs baseline: 4.3773x; 4.3773x over previous
import jax
import jax.numpy as jnp
from jax.experimental import pallas as pl

_T = 256


def _lrelu(x):
    return jnp.where(x >= 0, x, 0.01 * x)


def _first_max(y, k):
    maxv = jnp.max(y, axis=1, keepdims=True)
    li = jax.lax.broadcasted_iota(jnp.int32, y.shape, 1)
    return jnp.min(jnp.where(y == maxv, li, k), axis=1, keepdims=True)


def _select_expert(a, inds, group):
    ci = jax.lax.broadcasted_iota(jnp.int32, a.shape, 1) // group
    a = jnp.where(ci == inds, a, 0.0)
    while a.shape[1] > group:
        h = a.shape[1] // 2
        a = a[:, :h] + a[:, h:]
    return a


def _moe_kernel(x_ref, w1_ref, b1_ref, w2_ref, b2_ref, w3_ref, b3_ref,
                w21_ref, b21_ref, w22_ref, b22_ref, w23_ref, b23_ref,
                w31_ref, b31_ref, w32_ref, b32_ref, w33_ref, b33_ref,
                xr_ref, mask_ref):
    f32 = jnp.float32
    x = x_ref[...]
    h = _lrelu(jnp.dot(x, w1_ref[...], preferred_element_type=f32) + b1_ref[...])
    h = _lrelu(jnp.dot(h, w2_ref[...], preferred_element_type=f32) + b2_ref[...])
    y3 = jnp.dot(h, w3_ref[...], preferred_element_type=f32) + b3_ref[...]
    mask_ref[...] = _lrelu(y3[:, 16:17])
    inds1 = _first_max(y3[:, :16], 16)
    a = _lrelu(jnp.dot(x, w21_ref[...], preferred_element_type=f32) + b21_ref[...])
    g = _select_expert(a, inds1, 32)
    a = _lrelu(jnp.dot(g, w22_ref[...], preferred_element_type=f32) + b22_ref[...])
    g = _select_expert(a, inds1, 32)
    a = jnp.dot(g, w23_ref[...], preferred_element_type=f32) + b23_ref[...]
    x2 = _select_expert(a, inds1, 16)
    inds2 = _first_max(x2, 16)
    inds12 = inds1 * 16 + inds2
    a = _lrelu(jnp.dot(x, w31_ref[...], preferred_element_type=f32) + b31_ref[...])
    g = _select_expert(a, inds12, 32)
    a = _lrelu(jnp.dot(g, w32_ref[...], preferred_element_type=f32) + b32_ref[...])
    g = _select_expert(a, inds12, 16)
    a = jnp.dot(g, w33_ref[...], preferred_element_type=f32) + b33_ref[...]
    r = _select_expert(a, inds12, 1)
    xr_ref[...] = (inds12.astype(f32) + r) * (1.0 / 256.0)


def kernel(x_in, c1_1_w, c1_1_b, c1_2_w, c1_2_b, c1_3_w, c1_3_b,
           c2_1_w, c2_1_b, c2_2_w, c2_2_b, c2_3_w, c2_3_b,
           r1_1_w, r1_1_b, r1_2_w, r1_2_b, r1_3_w, r1_3_b):
    B, C, H, W = x_in.shape
    xf = jnp.transpose(x_in, (0, 3, 2, 1)).reshape(-1, C)
    n = xf.shape[0]
    f32 = jnp.float32

    params = (
        c1_1_w.T, c1_1_b.reshape(1, -1),
        c1_2_w.T, c1_2_b.reshape(1, -1),
        jnp.pad(c1_3_w.T, ((0, 0), (0, 15))), jnp.pad(c1_3_b, (0, 15)).reshape(1, -1),
        c2_1_w.transpose(1, 0, 2).reshape(C, -1), c2_1_b.reshape(1, -1),
        c2_2_w.transpose(1, 0, 2).reshape(32, -1), c2_2_b.reshape(1, -1),
        c2_3_w.transpose(1, 0, 2).reshape(32, -1), c2_3_b.reshape(1, -1),
        r1_1_w.transpose(1, 0, 2).reshape(C, -1), r1_1_b.reshape(1, -1),
        r1_2_w.transpose(1, 0, 2).reshape(32, -1), r1_2_b.reshape(1, -1),
        r1_3_w.transpose(1, 0, 2).reshape(16, -1), r1_3_b.reshape(1, -1),
    )

    def _const(shape):
        return pl.BlockSpec(shape, lambda i: (0, 0))

    in_specs = [pl.BlockSpec((_T, C), lambda i: (i, 0))]
    in_specs += [_const(p.shape) for p in params]
    out_specs = [pl.BlockSpec((_T, 1), lambda i: (i, 0)),
                 pl.BlockSpec((_T, 1), lambda i: (i, 0))]
    out_shape = (jax.ShapeDtypeStruct((n, 1), f32),
                 jax.ShapeDtypeStruct((n, 1), f32))

    xr, mask = pl.pallas_call(
        _moe_kernel,
        grid=(n // _T,),
        in_specs=in_specs,
        out_specs=out_specs,
        out_shape=out_shape,
    )(xf, *params)
    return xr.reshape(B, 1, H, W), mask.reshape(B, 1, H, W)


# speedup vs baseline: 5.3438x; 1.2208x over previous
import jax
import jax.numpy as jnp
from jax.experimental import pallas as pl

_T = 256


def _lrelu(x):
    return jnp.where(x >= 0, x, 0.01 * x)


def _first_max(y, k):
    maxv = jnp.max(y, axis=1, keepdims=True)
    li = jax.lax.broadcasted_iota(jnp.int32, y.shape, 1)
    return jnp.min(jnp.where(y == maxv, li, k), axis=1, keepdims=True)


def _select_expert(a, idrow, ind, group):
    a = jnp.where(idrow == ind, a, 0.0)
    while a.shape[1] > group:
        h = a.shape[1] // 2
        a = a[:, :h] + a[:, h:]
    return a


def _moe_kernel(x_ref, w1_ref, b1_ref, w2_ref, b2_ref, w3_ref, b3_ref,
                w21_ref, b21_ref, w22_ref, b22_ref, w23_ref, b23_ref,
                w31_ref, b31_ref, w32_ref, b32_ref, w33_ref, b33_ref,
                id21_ref, id23_ref, id31_ref, id32_ref, id33_ref,
                xr_ref, mask_ref):
    f32 = jnp.float32
    bf16 = jnp.bfloat16
    x = x_ref[...]
    h = _lrelu(jnp.dot(x, w1_ref[...], preferred_element_type=f32) + b1_ref[...])
    h = _lrelu(jnp.dot(h, w2_ref[...], preferred_element_type=f32) + b2_ref[...])
    y3 = jnp.dot(h, w3_ref[...], preferred_element_type=f32) + b3_ref[...]
    mask_ref[...] = _lrelu(y3[:, 16:17])
    inds1 = _first_max(y3[:, :16], 16)
    inds1f = inds1.astype(f32)
    a = jnp.dot(x, w21_ref[...], preferred_element_type=f32) + b21_ref[...]
    g = _lrelu(_select_expert(a, id21_ref[...], inds1f, 32))
    a = jnp.dot(g, w22_ref[...], preferred_element_type=f32) + b22_ref[...]
    g = _lrelu(_select_expert(a, id21_ref[...], inds1f, 32))
    a = jnp.dot(g, w23_ref[...], preferred_element_type=f32) + b23_ref[...]
    x2 = _select_expert(a, id23_ref[...], inds1f, 16)
    inds2 = _first_max(x2, 16)
    inds12 = inds1 * 16 + inds2
    inds12f = inds12.astype(f32)
    inds12h = inds12.astype(bf16)
    xh = x.astype(bf16)
    a = jnp.dot(xh, w31_ref[...], preferred_element_type=f32).astype(bf16) + b31_ref[...]
    g = _lrelu(_select_expert(a, id31_ref[...], inds12h, 32))
    a = jnp.dot(g, w32_ref[...], preferred_element_type=f32).astype(bf16) + b32_ref[...]
    g = _lrelu(_select_expert(a, id32_ref[...], inds12h, 16))
    a = jnp.dot(g, w33_ref[...], preferred_element_type=f32) + b33_ref[...]
    r = _select_expert(a, id33_ref[...], inds12f, 1)
    xr_ref[...] = (inds12f + r) * (1.0 / 256.0)


def kernel(x_in, c1_1_w, c1_1_b, c1_2_w, c1_2_b, c1_3_w, c1_3_b,
           c2_1_w, c2_1_b, c2_2_w, c2_2_b, c2_3_w, c2_3_b,
           r1_1_w, r1_1_b, r1_2_w, r1_2_b, r1_3_w, r1_3_b):
    B, C, H, W = x_in.shape
    xf = jnp.transpose(x_in, (0, 3, 2, 1)).reshape(-1, C)
    n = xf.shape[0]
    f32 = jnp.float32
    bf16 = jnp.bfloat16

    def _eid(width, group, dtype):
        return (jnp.arange(width, dtype=jnp.int32) // group).astype(dtype).reshape(1, width)

    params = (
        c1_1_w.T, c1_1_b.reshape(1, -1),
        c1_2_w.T, c1_2_b.reshape(1, -1),
        jnp.pad(c1_3_w.T, ((0, 0), (0, 15))), jnp.pad(c1_3_b, (0, 15)).reshape(1, -1),
        c2_1_w.transpose(1, 0, 2).reshape(C, -1), c2_1_b.reshape(1, -1),
        c2_2_w.transpose(1, 0, 2).reshape(32, -1), c2_2_b.reshape(1, -1),
        c2_3_w.transpose(1, 0, 2).reshape(32, -1), c2_3_b.reshape(1, -1),
        r1_1_w.transpose(1, 0, 2).reshape(C, -1).astype(bf16), r1_1_b.reshape(1, -1).astype(bf16),
        r1_2_w.transpose(1, 0, 2).reshape(32, -1).astype(bf16), r1_2_b.reshape(1, -1).astype(bf16),
        r1_3_w.transpose(1, 0, 2).reshape(16, -1).astype(bf16), r1_3_b.reshape(1, -1),
        _eid(512, 32, f32), _eid(256, 16, f32),
        _eid(8192, 32, bf16), _eid(4096, 16, bf16), _eid(256, 1, f32),
    )

    def _const(shape):
        return pl.BlockSpec(shape, lambda i: (0, 0))

    in_specs = [pl.BlockSpec((_T, C), lambda i: (i, 0))]
    in_specs += [_const(p.shape) for p in params]
    out_specs = [pl.BlockSpec((_T, 1), lambda i: (i, 0)),
                 pl.BlockSpec((_T, 1), lambda i: (i, 0))]
    out_shape = (jax.ShapeDtypeStruct((n, 1), f32),
                 jax.ShapeDtypeStruct((n, 1), f32))

    xr, mask = pl.pallas_call(
        _moe_kernel,
        grid=(n // _T,),
        in_specs=in_specs,
        out_specs=out_specs,
        out_shape=out_shape,
    )(xf, *params)
    return xr.reshape(B, 1, H, W), mask.reshape(B, 1, H, W)


# speedup vs baseline: 7.3153x; 1.3689x over previous
import jax
import jax.numpy as jnp
from jax.experimental import pallas as pl

_T = 512


def _lrelu(x):
    return jnp.where(x >= 0, x, 0.01 * x)


def _first_max(y, k):
    maxv = jnp.max(y, axis=1, keepdims=True)
    li = jax.lax.broadcasted_iota(jnp.int32, y.shape, 1)
    return jnp.min(jnp.where(y == maxv, li, k), axis=1, keepdims=True)


def _select_expert(a, idrow, ind, group):
    a = jnp.where(idrow == ind, a, 0.0)
    while a.shape[1] > group:
        h = a.shape[1] // 2
        a = a[:, :h] + a[:, h:]
    return a


def _moe_kernel(x_ref, w1_ref, b1_ref, w2_ref, b2_ref, w3_ref, b3_ref,
                w21_ref, b21_ref, w22_ref, b22_ref, w23_ref, b23_ref,
                w31_ref, b31_ref, w32_ref, b32_ref, w33_ref, b33_ref,
                id21_ref, id23_ref, id31_ref, id32_ref, id33_ref,
                xr_ref, mask_ref):
    f32 = jnp.float32
    bf16 = jnp.bfloat16
    x = x_ref[...]
    h = _lrelu(jnp.dot(x, w1_ref[...], preferred_element_type=f32) + b1_ref[...])
    h = _lrelu(jnp.dot(h, w2_ref[...], preferred_element_type=f32) + b2_ref[...])
    y3 = jnp.dot(h, w3_ref[...], preferred_element_type=f32) + b3_ref[...]
    mask_ref[...] = _lrelu(y3[:, 16:17])
    inds1 = _first_max(y3[:, :16], 16)
    inds1f = inds1.astype(f32)
    a = jnp.dot(x, w21_ref[...], preferred_element_type=f32) + b21_ref[...]
    g = _lrelu(_select_expert(a, id21_ref[...], inds1f, 32))
    a = jnp.dot(g, w22_ref[...], preferred_element_type=f32) + b22_ref[...]
    g = _lrelu(_select_expert(a, id21_ref[...], inds1f, 32))
    a = jnp.dot(g, w23_ref[...], preferred_element_type=f32) + b23_ref[...]
    x2 = _select_expert(a, id23_ref[...], inds1f, 16)
    inds2 = _first_max(x2, 16)
    inds12 = inds1 * 16 + inds2
    inds12f = inds12.astype(f32)
    inds12h = inds12.astype(bf16)
    onehot = jnp.where(id33_ref[...] == inds12f, 1.0, 0.0)
    xh = x.astype(bf16)
    a = jnp.dot(xh, w31_ref[...], preferred_element_type=f32).astype(bf16)
    g = _select_expert(a, id31_ref[...], inds12h, 32)
    bsel = jnp.dot(onehot, b31_ref[...], preferred_element_type=f32).astype(bf16)
    g = _lrelu(g + bsel)
    a = jnp.dot(g, w32_ref[...], preferred_element_type=f32).astype(bf16)
    g = _select_expert(a, id32_ref[...], inds12h, 16)
    bsel = jnp.dot(onehot, b32_ref[...], preferred_element_type=f32).astype(bf16)
    g = _lrelu(g + bsel)
    a = jnp.dot(g, w33_ref[...], preferred_element_type=f32) + b33_ref[...]
    r = jnp.sum(onehot * a, axis=1, keepdims=True)
    xr_ref[...] = (inds12f + r) * (1.0 / 256.0)


def kernel(x_in, c1_1_w, c1_1_b, c1_2_w, c1_2_b, c1_3_w, c1_3_b,
           c2_1_w, c2_1_b, c2_2_w, c2_2_b, c2_3_w, c2_3_b,
           r1_1_w, r1_1_b, r1_2_w, r1_2_b, r1_3_w, r1_3_b):
    B, C, H, W = x_in.shape
    xf = jnp.transpose(x_in, (0, 3, 2, 1)).reshape(-1, C)
    n = xf.shape[0]
    f32 = jnp.float32
    bf16 = jnp.bfloat16

    def _eid(width, group, dtype):
        return (jnp.arange(width, dtype=jnp.int32) // group).astype(dtype).reshape(1, width)

    params = (
        c1_1_w.T, c1_1_b.reshape(1, -1),
        c1_2_w.T, c1_2_b.reshape(1, -1),
        jnp.pad(c1_3_w.T, ((0, 0), (0, 15))), jnp.pad(c1_3_b, (0, 15)).reshape(1, -1),
        c2_1_w.transpose(1, 0, 2).reshape(C, -1), c2_1_b.reshape(1, -1),
        c2_2_w.transpose(1, 0, 2).reshape(32, -1), c2_2_b.reshape(1, -1),
        c2_3_w.transpose(1, 0, 2).reshape(32, -1), c2_3_b.reshape(1, -1),
        r1_1_w.transpose(1, 0, 2).reshape(C, -1).astype(bf16), r1_1_b,
        r1_2_w.transpose(1, 0, 2).reshape(32, -1).astype(bf16), r1_2_b,
        r1_3_w.transpose(1, 0, 2).reshape(16, -1).astype(bf16), r1_3_b.reshape(1, -1),
        _eid(512, 32, f32), _eid(256, 16, f32),
        _eid(8192, 32, bf16), _eid(4096, 16, bf16), _eid(256, 1, f32),
    )

    def _const(shape):
        return pl.BlockSpec(shape, lambda i: (0, 0))

    in_specs = [pl.BlockSpec((_T, C), lambda i: (i, 0))]
    in_specs += [_const(p.shape) for p in params]
    out_specs = [pl.BlockSpec((_T, 1), lambda i: (i, 0)),
                 pl.BlockSpec((_T, 1), lambda i: (i, 0))]
    out_shape = (jax.ShapeDtypeStruct((n, 1), f32),
                 jax.ShapeDtypeStruct((n, 1), f32))

    xr, mask = pl.pallas_call(
        _moe_kernel,
        grid=(n // _T,),
        in_specs=in_specs,
        out_specs=out_specs,
        out_shape=out_shape,
    )(xf, *params)
    return xr.reshape(B, 1, H, W), mask.reshape(B, 1, H, W)
